# baseline (speedup 1.0000x reference)
import jax
import jax.numpy as jnp
from jax import lax
from jax.experimental import pallas as pl
from jax.experimental.pallas import tpu as pltpu

B = 2
C = 64
H = 256
G = 64
N = G * G
EPS = 1e-6
C_CHUNK = 8
BLK = 512
WIN = 768
OFFSETS = (1, -1, G, -G)


def _weights(xv_ref, w_ref):
    xv = xv_ref[0]
    ci = lax.broadcasted_iota(jnp.int32, (1, N), 1) % G
    first_col = ci == 0
    last_col = ci == G - 1

    def sh(a, off):
        if off < 0:
            return jnp.concatenate([a[:, :-off], a[:, :off]], axis=1)
        return jnp.concatenate([a[:, off:], a[:, -off:]], axis=1)

    up = sh(xv, -G)
    dn = sh(xv, G)
    p1 = 0.125 * up + 0.75 * xv + 0.125 * dn
    lf = jnp.where(first_col, p1, sh(p1, -1))
    rt = jnp.where(last_col, p1, sh(p1, 1))
    p2 = 0.125 * lf + 0.75 * p1 + 0.125 * rt
    um = 1.0 - p2

    def t(x):
        return jnp.where(x > EPS, x, 0.0)

    w_ref[0:1, :] = jnp.where(last_col, 0.0, t(sh(um, 1) - um))
    w_ref[1:2, :] = jnp.where(first_col, 0.0, t(sh(um, -1) - um))
    w_ref[2:3, :] = t(sh(um, G) - um)
    w_ref[3:4, :] = t(sh(um, -G) - um)


def _fused(xf_ref, xv_ref, nodes_ref, adj_ref, acc_ref, w_ref):
    k = pl.program_id(1)

    @pl.when(k == 0)
    def _init():
        acc_ref[...] = jnp.zeros_like(acc_ref)
        _weights(xv_ref, w_ref)

    acc_ref[...] += jnp.sum(xf_ref[0], axis=0)

    adj_ref[...] = jnp.zeros_like(adj_ref)
    start = pl.multiple_of(jnp.clip(BLK * k - 128, 0, N - WIN), 128)
    rowi = BLK * k + lax.broadcasted_iota(jnp.int32, (BLK, WIN), 0)
    coli = start + lax.broadcasted_iota(jnp.int32, (BLK, WIN), 1)
    delta = rowi - coli
    band = jnp.zeros((BLK, WIN), jnp.float32)
    for d, offs in enumerate(OFFSETS):
        wv = w_ref[d, pl.ds(start, WIN)]
        band = jnp.where(delta == offs, wv[None, :], band)
    adj_ref[0, :, pl.ds(start, WIN)] = band

    @pl.when(k == pl.num_programs(1) - 1)
    def _fin():
        y = acc_ref[...]
        a = lax.broadcasted_iota(jnp.int32, (G, H), 0)
        b = lax.broadcasted_iota(jnp.int32, (G, H), 1) // 4
        P = jnp.where(a == b, 0.25, 0.0).astype(jnp.float32)
        s = jax.lax.dot_general(
            jax.lax.dot_general(P, y, (((1,), (0,)), ((), ())),
                                precision=lax.Precision.HIGHEST),
            P, (((1,), (1,)), ((), ())),
            precision=lax.Precision.HIGHEST)
        for t in range(G):
            nodes_ref[0, G * t:G * (t + 1), :] = s


def kernel(x_feat, x_var):
    xv_flat = x_var.reshape(B, 1, N)
    nodes, adjs = pl.pallas_call(
        _fused,
        grid=(B, N // BLK),
        in_specs=[
            pl.BlockSpec((1, C_CHUNK, H, H), lambda b, k: (b, k, 0, 0)),
            pl.BlockSpec((1, 1, N), lambda b, k: (b, 0, 0)),
        ],
        out_specs=[
            pl.BlockSpec((1, N, C), lambda b, k: (b, 0, 0)),
            pl.BlockSpec((1, BLK, N), lambda b, k: (b, k, 0)),
        ],
        out_shape=[
            jax.ShapeDtypeStruct((B, N, C), jnp.float32),
            jax.ShapeDtypeStruct((B, N, N), jnp.float32),
        ],
        scratch_shapes=[
            pltpu.VMEM((H, H), jnp.float32),
            pltpu.VMEM((4, N), jnp.float32),
        ],
        compiler_params=pltpu.CompilerParams(
            dimension_semantics=("parallel", "arbitrary")),
    )(x_feat, xv_flat)
    return nodes, adjs


# speedup vs baseline: 1.0160x; 1.0160x over previous
import jax
import jax.numpy as jnp
from jax import lax
from jax.experimental import pallas as pl
from jax.experimental.pallas import tpu as pltpu

B = 2
C = 64
H = 256
G = 64
N = G * G
EPS = 1e-6
C_CHUNK = 16
BLK = 1024
WIN = 1280
OFFSETS = (1, -1, G, -G)


def _weights(xv_ref, w_ref):
    xv = xv_ref[0]
    ci = lax.broadcasted_iota(jnp.int32, (1, N), 1) % G
    first_col = ci == 0
    last_col = ci == G - 1

    def sh(a, off):
        if off < 0:
            return jnp.concatenate([a[:, :-off], a[:, :off]], axis=1)
        return jnp.concatenate([a[:, off:], a[:, -off:]], axis=1)

    up = sh(xv, -G)
    dn = sh(xv, G)
    p1 = 0.125 * up + 0.75 * xv + 0.125 * dn
    lf = jnp.where(first_col, p1, sh(p1, -1))
    rt = jnp.where(last_col, p1, sh(p1, 1))
    p2 = 0.125 * lf + 0.75 * p1 + 0.125 * rt
    um = 1.0 - p2

    def t(x):
        return jnp.where(x > EPS, x, 0.0)

    w_ref[0:1, :] = jnp.where(last_col, 0.0, t(sh(um, 1) - um))
    w_ref[1:2, :] = jnp.where(first_col, 0.0, t(sh(um, -1) - um))
    w_ref[2:3, :] = t(sh(um, G) - um)
    w_ref[3:4, :] = t(sh(um, -G) - um)


def _fused(xf_ref, xv_ref, nodes_ref, adj_ref, acc_ref, w_ref):
    k = pl.program_id(1)

    @pl.when(k == 0)
    def _init():
        acc_ref[...] = jnp.zeros_like(acc_ref)
        _weights(xv_ref, w_ref)

    acc_ref[...] += jnp.sum(xf_ref[0], axis=0)

    adj_ref[...] = jnp.zeros_like(adj_ref)
    start = pl.multiple_of(jnp.clip(BLK * k - 128, 0, N - WIN), 128)
    rowi = BLK * k + lax.broadcasted_iota(jnp.int32, (BLK, WIN), 0)
    coli = start + lax.broadcasted_iota(jnp.int32, (BLK, WIN), 1)
    delta = rowi - coli
    band = jnp.zeros((BLK, WIN), jnp.float32)
    for d, offs in enumerate(OFFSETS):
        wv = w_ref[d, pl.ds(start, WIN)]
        band = jnp.where(delta == offs, wv[None, :], band)
    adj_ref[0, :, pl.ds(start, WIN)] = band

    @pl.when(k == pl.num_programs(1) - 1)
    def _fin():
        y = acc_ref[...]
        a = lax.broadcasted_iota(jnp.int32, (G, H), 0)
        b = lax.broadcasted_iota(jnp.int32, (G, H), 1) // 4
        P = jnp.where(a == b, 0.25, 0.0).astype(jnp.float32)
        s = jax.lax.dot_general(
            jax.lax.dot_general(P, y, (((1,), (0,)), ((), ())),
                                precision=lax.Precision.HIGHEST),
            P, (((1,), (1,)), ((), ())),
            precision=lax.Precision.HIGHEST)
        for t in range(G):
            nodes_ref[0, G * t:G * (t + 1), :] = s


def kernel(x_feat, x_var):
    xv_flat = x_var.reshape(B, 1, N)
    nodes, adjs = pl.pallas_call(
        _fused,
        grid=(B, N // BLK),
        in_specs=[
            pl.BlockSpec((1, C_CHUNK, H, H), lambda b, k: (b, k, 0, 0)),
            pl.BlockSpec((1, 1, N), lambda b, k: (b, 0, 0)),
        ],
        out_specs=[
            pl.BlockSpec((1, N, C), lambda b, k: (b, 0, 0)),
            pl.BlockSpec((1, BLK, N), lambda b, k: (b, k, 0)),
        ],
        out_shape=[
            jax.ShapeDtypeStruct((B, N, C), jnp.float32),
            jax.ShapeDtypeStruct((B, N, N), jnp.float32),
        ],
        scratch_shapes=[
            pltpu.VMEM((H, H), jnp.float32),
            pltpu.VMEM((4, N), jnp.float32),
        ],
        compiler_params=pltpu.CompilerParams(
            dimension_semantics=("parallel", "arbitrary")),
    )(x_feat, xv_flat)
    return nodes, adjs
